# Initial kernel scaffold; baseline (speedup 1.0000x reference)
import functools

import jax
import jax.numpy as jnp
from jax import lax
from jax.experimental import pallas as pl
from jax.experimental.pallas import tpu as pltpu
from jax.experimental.pallas import tpu_sc as plsc

N = 10000
E = 320000
IN_FEATS = 128
HID = 32

NC = 2
NS = 16
NW = NC * NS
CH = 128
CPT = 79
EPT = CH * CPT
E_PAD = NW * EPT
N_PAD = 10016
ROWS_PER_TILE = N_PAD // NS

_mesh = plsc.VectorSubcoreMesh(core_axis_name="c", subcore_axis_name="s")


@functools.partial(
    pl.kernel,
    mesh=_mesh,
    out_type=jax.ShapeDtypeStruct((NC, N_PAD, 2), jnp.float32),
    scratch_types=[
        pltpu.VMEM((CPT, CH), jnp.int32),
        pltpu.VMEM((CPT, CH), jnp.int32),
        pltpu.VMEM((CH, 2), jnp.float32),
        pltpu.VMEM((CH, 2), jnp.float32),
        pltpu.VMEM_SHARED((N_PAD, 2), jnp.float32),
    ],
)
def _sc_degrees(src_hbm, dst_hbm, zeros2_hbm, e10_hbm, e01_hbm, out_hbm,
                srcv, dstv, e10_v, e01_v, deg_sh):
    cid = lax.axis_index("c")
    sid = lax.axis_index("s")
    wid = sid * NC + cid
    r0 = sid * ROWS_PER_TILE
    pltpu.sync_copy(zeros2_hbm.at[pl.ds(r0, ROWS_PER_TILE)],
                    deg_sh.at[pl.ds(r0, ROWS_PER_TILE)])
    pltpu.sync_copy(e10_hbm, e10_v)
    pltpu.sync_copy(e01_hbm, e01_v)
    pltpu.sync_copy(src_hbm.at[wid], srcv)
    pltpu.sync_copy(dst_hbm.at[wid], dstv)
    plsc.subcore_barrier()

    def chunk(j, c):
        pltpu.sync_copy(e10_v, deg_sh.at[srcv.at[j]], add=True)
        pltpu.sync_copy(e01_v, deg_sh.at[dstv.at[j]], add=True)
        return c

    lax.fori_loop(0, CPT, chunk, 0)
    plsc.subcore_barrier()
    pltpu.sync_copy(deg_sh.at[pl.ds(r0, ROWS_PER_TILE)],
                    out_hbm.at[cid, pl.ds(r0, ROWS_PER_TILE)])


@functools.partial(
    pl.kernel,
    mesh=_mesh,
    out_type=jax.ShapeDtypeStruct((NC, N_PAD, HID), jnp.float32),
    scratch_types=[
        pltpu.VMEM((CPT, CH), jnp.int32),
        pltpu.VMEM((CPT, CH), jnp.int32),
        pltpu.VMEM((CH, HID), jnp.float32),
        pltpu.VMEM_SHARED((N_PAD, HID), jnp.float32),
        pltpu.VMEM_SHARED((N_PAD, HID), jnp.float32),
    ],
)
def _sc_message_pass(h_hbm, src_hbm, dst_hbm, zeros_hbm, out_hbm,
                     srcv, dstv, rows_v, h_sh, agg_sh):
    cid = lax.axis_index("c")
    sid = lax.axis_index("s")
    wid = sid * NC + cid
    r0 = sid * ROWS_PER_TILE
    pltpu.sync_copy(h_hbm.at[pl.ds(r0, ROWS_PER_TILE)],
                    h_sh.at[pl.ds(r0, ROWS_PER_TILE)])
    pltpu.sync_copy(zeros_hbm.at[pl.ds(r0, ROWS_PER_TILE)],
                    agg_sh.at[pl.ds(r0, ROWS_PER_TILE)])
    pltpu.sync_copy(src_hbm.at[wid], srcv)
    pltpu.sync_copy(dst_hbm.at[wid], dstv)
    plsc.subcore_barrier()

    def chunk(j, c):
        pltpu.sync_copy(h_sh.at[srcv.at[j]], rows_v)
        pltpu.sync_copy(rows_v, agg_sh.at[dstv.at[j]], add=True)
        return c

    lax.fori_loop(0, CPT, chunk, 0)
    plsc.subcore_barrier()
    pltpu.sync_copy(agg_sh.at[pl.ds(r0, ROWS_PER_TILE)],
                    out_hbm.at[cid, pl.ds(r0, ROWS_PER_TILE)])


RB = 1000


def _tc1_body(x_ref, degp_ref, w_ref, h_ref, norm_ref):
    d = degp_ref[...]
    deg = d[0] + d[1]
    norm = jnp.where(deg > 0.0, lax.rsqrt(jnp.maximum(deg, 1.0)), 0.0)
    norm_ref[...] = norm
    xs = x_ref[...] * norm[:, 0:1]
    h_ref[...] = jnp.dot(xs, w_ref[...], preferred_element_type=jnp.float32)


def _tc2_body(p0_ref, p1_ref, norm_ref, b1_ref, w2_ref, h2_ref):
    norm = norm_ref[...]
    h = (p0_ref[...] + p1_ref[...]) * norm[:, 1:2] + b1_ref[...]
    h = jnp.maximum(h, 0.0)
    h = h * norm[:, 0:1]
    h2_ref[...] = jnp.dot(h, w2_ref[...], preferred_element_type=jnp.float32)


def _tc3_body(p0_ref, p1_ref, norm_ref, b2_ref, o_ref):
    o_ref[...] = ((p0_ref[...] + p1_ref[...]) * norm_ref[...][:, 1:2]
                  + b2_ref[...])


_G = N // RB

_tc1 = pl.pallas_call(
    _tc1_body,
    grid=(_G,),
    in_specs=[
        pl.BlockSpec((RB, IN_FEATS), lambda i: (i, 0)),
        pl.BlockSpec((NC, RB, 2), lambda i: (0, i, 0)),
        pl.BlockSpec((IN_FEATS, HID), lambda i: (0, 0)),
    ],
    out_specs=[
        pl.BlockSpec((RB, HID), lambda i: (i, 0)),
        pl.BlockSpec((RB, 2), lambda i: (i, 0)),
    ],
    out_shape=[
        jax.ShapeDtypeStruct((N, HID), jnp.float32),
        jax.ShapeDtypeStruct((N, 2), jnp.float32),
    ],
)

_tc2 = pl.pallas_call(
    _tc2_body,
    grid=(_G,),
    in_specs=[
        pl.BlockSpec((RB, HID), lambda i: (i, 0)),
        pl.BlockSpec((RB, HID), lambda i: (i, 0)),
        pl.BlockSpec((RB, 2), lambda i: (i, 0)),
        pl.BlockSpec((1, HID), lambda i: (0, 0)),
        pl.BlockSpec((HID, HID), lambda i: (0, 0)),
    ],
    out_specs=pl.BlockSpec((RB, HID), lambda i: (i, 0)),
    out_shape=jax.ShapeDtypeStruct((N, HID), jnp.float32),
)

_tc3 = pl.pallas_call(
    _tc3_body,
    grid=(_G,),
    in_specs=[
        pl.BlockSpec((RB, HID), lambda i: (i, 0)),
        pl.BlockSpec((RB, HID), lambda i: (i, 0)),
        pl.BlockSpec((RB, 2), lambda i: (i, 0)),
        pl.BlockSpec((1, HID), lambda i: (0, 0)),
    ],
    out_specs=pl.BlockSpec((RB, HID), lambda i: (i, 0)),
    out_shape=jax.ShapeDtypeStruct((N, HID), jnp.float32),
)


def kernel(x, edge_index, W1, b1, W2, b2):
    src = edge_index[0]
    dst = edge_index[1]
    pad_idx = (jnp.arange(E_PAD - E, dtype=jnp.int32) % (N_PAD - N)) + N
    src_p = jnp.concatenate([src, pad_idx]).reshape(NW, CPT, CH)
    dst_p = jnp.concatenate([dst, pad_idx]).reshape(NW, CPT, CH)

    zeros2 = jnp.zeros((N_PAD, 2), jnp.float32)
    zeros_f = jnp.zeros((N_PAD, HID), jnp.float32)
    e10 = jnp.tile(jnp.array([[1.0, 0.0]], jnp.float32), (CH, 1))
    e01 = jnp.tile(jnp.array([[0.0, 1.0]], jnp.float32), (CH, 1))

    degp = _sc_degrees(src_p, dst_p, zeros2, e10, e01)
    h1, norms = _tc1(x, degp[:, :N, :], W1)

    h1p = jnp.pad(h1, ((0, N_PAD - N), (0, 0)))
    agg1 = _sc_message_pass(h1p, src_p, dst_p, zeros_f)
    h2 = _tc2(agg1[0, :N], agg1[1, :N], norms,
              b1.reshape(1, HID), W2)

    h2p = jnp.pad(h2, ((0, N_PAD - N), (0, 0)))
    agg2 = _sc_message_pass(h2p, src_p, dst_p, zeros_f)
    out = _tc3(agg2[0, :N], agg2[1, :N], norms, b2.reshape(1, HID))
    return out


# speedup vs baseline: 7.7898x; 7.7898x over previous
import functools

import jax
import jax.numpy as jnp
from jax import lax
from jax.experimental import pallas as pl
from jax.experimental.pallas import tpu as pltpu
from jax.experimental.pallas import tpu_sc as plsc

N = 10000
E = 320000
IN_FEATS = 128
HID = 32

NC = 2
NS = 16
NW = NC * NS
CH = 128
CPT = 79
EPT = CH * CPT
E_PAD = NW * EPT
N_PAD = 10240
ROWS_PER_TILE = N_PAD // NS
STAGE_ROWS = 160
N_SUB = ROWS_PER_TILE // STAGE_ROWS

_mesh = plsc.VectorSubcoreMesh(core_axis_name="c", subcore_axis_name="s")


@functools.partial(
    pl.kernel,
    mesh=_mesh,
    compiler_params=pltpu.CompilerParams(use_tc_tiling_on_sc=False),
    out_type=jax.ShapeDtypeStruct((NC, N_PAD, HID), jnp.float32),
    scratch_types=[
        pltpu.VMEM((CPT, CH), jnp.int32),
        pltpu.VMEM((CPT, CH), jnp.int32),
        pltpu.VMEM((CH, HID), jnp.float32),
        pltpu.VMEM((STAGE_ROWS, HID), jnp.float32),
        pltpu.VMEM_SHARED((N_PAD, HID), jnp.float32),
    ],
)
def _sc_message_pass(h_hbm, src_hbm, dst_hbm, zeros_hbm, out_hbm,
                     srcv, dstv, rows_v, stage_v, agg_sh):
    cid = lax.axis_index("c")
    sid = lax.axis_index("s")
    wid = sid * NC + cid
    r0 = sid * ROWS_PER_TILE
    pltpu.sync_copy(zeros_hbm, stage_v)
    for k in range(N_SUB):
        pltpu.sync_copy(stage_v,
                        agg_sh.at[pl.ds(r0 + k * STAGE_ROWS, STAGE_ROWS)])
    pltpu.sync_copy(src_hbm.at[wid], srcv)
    pltpu.sync_copy(dst_hbm.at[wid], dstv)
    plsc.subcore_barrier()

    def chunk(j, c):
        pltpu.sync_copy(h_hbm.at[srcv.at[j]], rows_v)
        pltpu.sync_copy(rows_v, agg_sh.at[dstv.at[j]], add=True)
        return c

    lax.fori_loop(0, CPT, chunk, 0)
    plsc.subcore_barrier()
    for k in range(N_SUB):
        sub = pl.ds(r0 + k * STAGE_ROWS, STAGE_ROWS)
        pltpu.sync_copy(agg_sh.at[sub], stage_v)
        pltpu.sync_copy(stage_v, out_hbm.at[cid, sub])


RB = 1000


def _tc1_body(x_ref, do0_ref, do1_ref, di0_ref, di1_ref, w_ref,
              h_ref, norm_ref):
    dout = do0_ref[...] + do1_ref[...]
    din = di0_ref[...] + di1_ref[...]
    nsrc = jnp.where(dout > 0.0, lax.rsqrt(jnp.maximum(dout, 1.0)), 0.0)
    ndst = jnp.where(din > 0.0, lax.rsqrt(jnp.maximum(din, 1.0)), 0.0)
    norm_ref[...] = jnp.concatenate([nsrc, ndst], axis=1)
    xs = x_ref[...] * nsrc
    h_ref[...] = jnp.dot(xs, w_ref[...], preferred_element_type=jnp.float32)


def _tc2_body(p0_ref, p1_ref, norm_ref, b1_ref, w2_ref, h2_ref):
    norm = norm_ref[...]
    h = (p0_ref[...] + p1_ref[...]) * norm[:, 1:2] + b1_ref[...]
    h = jnp.maximum(h, 0.0)
    h = h * norm[:, 0:1]
    h2_ref[...] = jnp.dot(h, w2_ref[...], preferred_element_type=jnp.float32)


def _tc3_body(p0_ref, p1_ref, norm_ref, b2_ref, o_ref):
    o_ref[...] = ((p0_ref[...] + p1_ref[...]) * norm_ref[...][:, 1:2]
                  + b2_ref[...])


_G = N // RB

_tc1 = pl.pallas_call(
    _tc1_body,
    grid=(_G,),
    in_specs=[
        pl.BlockSpec((RB, IN_FEATS), lambda i: (i, 0)),
        pl.BlockSpec((RB, 1), lambda i: (i, 0)),
        pl.BlockSpec((RB, 1), lambda i: (i, 0)),
        pl.BlockSpec((RB, 1), lambda i: (i, 0)),
        pl.BlockSpec((RB, 1), lambda i: (i, 0)),
        pl.BlockSpec((IN_FEATS, HID), lambda i: (0, 0)),
    ],
    out_specs=[
        pl.BlockSpec((RB, HID), lambda i: (i, 0)),
        pl.BlockSpec((RB, 2), lambda i: (i, 0)),
    ],
    out_shape=[
        jax.ShapeDtypeStruct((N, HID), jnp.float32),
        jax.ShapeDtypeStruct((N, 2), jnp.float32),
    ],
)

_tc2 = pl.pallas_call(
    _tc2_body,
    grid=(_G,),
    in_specs=[
        pl.BlockSpec((RB, HID), lambda i: (i, 0)),
        pl.BlockSpec((RB, HID), lambda i: (i, 0)),
        pl.BlockSpec((RB, 2), lambda i: (i, 0)),
        pl.BlockSpec((1, HID), lambda i: (0, 0)),
        pl.BlockSpec((HID, HID), lambda i: (0, 0)),
    ],
    out_specs=pl.BlockSpec((RB, HID), lambda i: (i, 0)),
    out_shape=jax.ShapeDtypeStruct((N, HID), jnp.float32),
)

_tc3 = pl.pallas_call(
    _tc3_body,
    grid=(_G,),
    in_specs=[
        pl.BlockSpec((RB, HID), lambda i: (i, 0)),
        pl.BlockSpec((RB, HID), lambda i: (i, 0)),
        pl.BlockSpec((RB, 2), lambda i: (i, 0)),
        pl.BlockSpec((1, HID), lambda i: (0, 0)),
    ],
    out_specs=pl.BlockSpec((RB, HID), lambda i: (i, 0)),
    out_shape=jax.ShapeDtypeStruct((N, HID), jnp.float32),
)


def kernel(x, edge_index, W1, b1, W2, b2):
    src = edge_index[0]
    dst = edge_index[1]
    pad_idx = (jnp.arange(E_PAD - E, dtype=jnp.int32) % (N_PAD - N)) + N
    src_p = jnp.concatenate([src, pad_idx]).reshape(NW, CPT, CH)
    dst_p = jnp.concatenate([dst, pad_idx]).reshape(NW, CPT, CH)

    zeros_f = jnp.zeros((STAGE_ROWS, HID), jnp.float32)
    ones_t = jnp.ones((N_PAD, HID), jnp.float32)

    din_f = _sc_message_pass(ones_t, src_p, dst_p, zeros_f)
    dout_f = _sc_message_pass(ones_t, dst_p, src_p, zeros_f)

    h1, norms = _tc1(x,
                     dout_f[0, :N, 0:1], dout_f[1, :N, 0:1],
                     din_f[0, :N, 0:1], din_f[1, :N, 0:1],
                     W1)

    h1p = jnp.pad(h1, ((0, N_PAD - N), (0, 0)))
    agg1 = _sc_message_pass(h1p, src_p, dst_p, zeros_f)
    h2 = _tc2(agg1[0, :N], agg1[1, :N], norms, b1.reshape(1, HID), W2)

    h2p = jnp.pad(h2, ((0, N_PAD - N), (0, 0)))
    agg2 = _sc_message_pass(h2p, src_p, dst_p, zeros_f)
    out = _tc3(agg2[0, :N], agg2[1, :N], norms, b2.reshape(1, HID))
    return out


# speedup vs baseline: 11.0877x; 1.4234x over previous
import functools

import jax
import jax.numpy as jnp
from jax import lax
from jax.experimental import pallas as pl
from jax.experimental.pallas import tpu as pltpu
from jax.experimental.pallas import tpu_sc as plsc

N = 10000
E = 320000
IN_FEATS = 128
HID = 32

NC = 2
NS = 16
NW = NC * NS
CH = 128
CPT = 79
EPT = CH * CPT
E_PAD = NW * EPT
N_PAD = 10240
ROWS_PER_TILE = N_PAD // NS
STAGE_ROWS = 160
N_SUB = ROWS_PER_TILE // STAGE_ROWS

_mesh = plsc.VectorSubcoreMesh(core_axis_name="c", subcore_axis_name="s")


@functools.partial(
    pl.kernel,
    mesh=_mesh,
    compiler_params=pltpu.CompilerParams(use_tc_tiling_on_sc=False),
    out_type=jax.ShapeDtypeStruct((NC, N_PAD, HID), jnp.float32),
    scratch_types=[
        pltpu.VMEM((CPT, CH), jnp.int32),
        pltpu.VMEM((CPT, CH), jnp.int32),
        pltpu.VMEM((CH, HID), jnp.float32),
        pltpu.VMEM((CH, HID), jnp.float32),
        pltpu.VMEM((STAGE_ROWS, HID), jnp.float32),
        pltpu.VMEM_SHARED((N_PAD, HID), jnp.float32),
        pltpu.SemaphoreType.DMA,
        pltpu.SemaphoreType.DMA,
    ],
)
def _sc_message_pass(h_hbm, src_hbm, dst_hbm, zeros_hbm, out_hbm,
                     srcv, dstv, rows_a, rows_b, stage_v, agg_sh,
                     sem_a, sem_b):
    cid = lax.axis_index("c")
    sid = lax.axis_index("s")
    wid = sid * NC + cid
    r0 = sid * ROWS_PER_TILE
    pltpu.sync_copy(zeros_hbm, stage_v)
    for k in range(N_SUB):
        pltpu.sync_copy(stage_v,
                        agg_sh.at[pl.ds(r0 + k * STAGE_ROWS, STAGE_ROWS)])
    pltpu.sync_copy(src_hbm.at[wid], srcv)
    pltpu.sync_copy(dst_hbm.at[wid], dstv)
    plsc.subcore_barrier()

    pltpu.make_async_copy(h_hbm.at[srcv.at[0]], rows_a, sem_a).start()

    def pair(i, c):
        j = 1 + 2 * i
        pltpu.make_async_copy(h_hbm.at[srcv.at[j]], rows_b, sem_b).start()
        pltpu.make_async_copy(h_hbm.at[srcv.at[j - 1]], rows_a, sem_a).wait()
        pltpu.sync_copy(rows_a, agg_sh.at[dstv.at[j - 1]], add=True)
        pltpu.make_async_copy(h_hbm.at[srcv.at[j + 1]], rows_a, sem_a).start()
        pltpu.make_async_copy(h_hbm.at[srcv.at[j]], rows_b, sem_b).wait()
        pltpu.sync_copy(rows_b, agg_sh.at[dstv.at[j]], add=True)
        return c

    lax.fori_loop(0, (CPT - 1) // 2, pair, 0)
    pltpu.make_async_copy(h_hbm.at[srcv.at[CPT - 1]], rows_a, sem_a).wait()
    pltpu.sync_copy(rows_a, agg_sh.at[dstv.at[CPT - 1]], add=True)
    plsc.subcore_barrier()
    for k in range(N_SUB):
        sub = pl.ds(r0 + k * STAGE_ROWS, STAGE_ROWS)
        pltpu.sync_copy(agg_sh.at[sub], stage_v)
        pltpu.sync_copy(stage_v, out_hbm.at[cid, sub])


RB = 1000


def _tc1_body(x_ref, do0_ref, do1_ref, di0_ref, di1_ref, w_ref,
              h_ref, norm_ref):
    dout = do0_ref[...] + do1_ref[...]
    din = di0_ref[...] + di1_ref[...]
    nsrc = jnp.where(dout > 0.0, lax.rsqrt(jnp.maximum(dout, 1.0)), 0.0)
    ndst = jnp.where(din > 0.0, lax.rsqrt(jnp.maximum(din, 1.0)), 0.0)
    norm_ref[...] = jnp.concatenate([nsrc, ndst], axis=1)
    xs = x_ref[...] * nsrc
    h_ref[...] = jnp.dot(xs, w_ref[...], preferred_element_type=jnp.float32)


def _tc2_body(p0_ref, p1_ref, norm_ref, b1_ref, w2_ref, h2_ref):
    norm = norm_ref[...]
    h = (p0_ref[...] + p1_ref[...]) * norm[:, 1:2] + b1_ref[...]
    h = jnp.maximum(h, 0.0)
    h = h * norm[:, 0:1]
    h2_ref[...] = jnp.dot(h, w2_ref[...], preferred_element_type=jnp.float32)


def _tc3_body(p0_ref, p1_ref, norm_ref, b2_ref, o_ref):
    o_ref[...] = ((p0_ref[...] + p1_ref[...]) * norm_ref[...][:, 1:2]
                  + b2_ref[...])


_G = N // RB

_tc1 = pl.pallas_call(
    _tc1_body,
    grid=(_G,),
    in_specs=[
        pl.BlockSpec((RB, IN_FEATS), lambda i: (i, 0)),
        pl.BlockSpec((RB, 1), lambda i: (i, 0)),
        pl.BlockSpec((RB, 1), lambda i: (i, 0)),
        pl.BlockSpec((RB, 1), lambda i: (i, 0)),
        pl.BlockSpec((RB, 1), lambda i: (i, 0)),
        pl.BlockSpec((IN_FEATS, HID), lambda i: (0, 0)),
    ],
    out_specs=[
        pl.BlockSpec((RB, HID), lambda i: (i, 0)),
        pl.BlockSpec((RB, 2), lambda i: (i, 0)),
    ],
    out_shape=[
        jax.ShapeDtypeStruct((N, HID), jnp.float32),
        jax.ShapeDtypeStruct((N, 2), jnp.float32),
    ],
)

_tc2 = pl.pallas_call(
    _tc2_body,
    grid=(_G,),
    in_specs=[
        pl.BlockSpec((RB, HID), lambda i: (i, 0)),
        pl.BlockSpec((RB, HID), lambda i: (i, 0)),
        pl.BlockSpec((RB, 2), lambda i: (i, 0)),
        pl.BlockSpec((1, HID), lambda i: (0, 0)),
        pl.BlockSpec((HID, HID), lambda i: (0, 0)),
    ],
    out_specs=pl.BlockSpec((RB, HID), lambda i: (i, 0)),
    out_shape=jax.ShapeDtypeStruct((N, HID), jnp.float32),
)

_tc3 = pl.pallas_call(
    _tc3_body,
    grid=(_G,),
    in_specs=[
        pl.BlockSpec((RB, HID), lambda i: (i, 0)),
        pl.BlockSpec((RB, HID), lambda i: (i, 0)),
        pl.BlockSpec((RB, 2), lambda i: (i, 0)),
        pl.BlockSpec((1, HID), lambda i: (0, 0)),
    ],
    out_specs=pl.BlockSpec((RB, HID), lambda i: (i, 0)),
    out_shape=jax.ShapeDtypeStruct((N, HID), jnp.float32),
)


def kernel(x, edge_index, W1, b1, W2, b2):
    src = edge_index[0]
    dst = edge_index[1]
    pad_idx = (jnp.arange(E_PAD - E, dtype=jnp.int32) % (N_PAD - N)) + N
    src_p = jnp.concatenate([src, pad_idx]).reshape(NW, CPT, CH)
    dst_p = jnp.concatenate([dst, pad_idx]).reshape(NW, CPT, CH)

    zeros_f = jnp.zeros((STAGE_ROWS, HID), jnp.float32)
    ones_t = jnp.ones((N_PAD, HID), jnp.float32)

    din_f = _sc_message_pass(ones_t, src_p, dst_p, zeros_f)
    dout_f = _sc_message_pass(ones_t, dst_p, src_p, zeros_f)

    h1, norms = _tc1(x,
                     dout_f[0, :N, 0:1], dout_f[1, :N, 0:1],
                     din_f[0, :N, 0:1], din_f[1, :N, 0:1],
                     W1)

    h1p = jnp.pad(h1, ((0, N_PAD - N), (0, 0)))
    agg1 = _sc_message_pass(h1p, src_p, dst_p, zeros_f)
    h2 = _tc2(agg1[0, :N], agg1[1, :N], norms, b1.reshape(1, HID), W2)

    h2p = jnp.pad(h2, ((0, N_PAD - N), (0, 0)))
    agg2 = _sc_message_pass(h2p, src_p, dst_p, zeros_f)
    out = _tc3(agg2[0, :N], agg2[1, :N], norms, b2.reshape(1, HID))
    return out


# speedup vs baseline: 12.1604x; 1.0967x over previous
import functools

import jax
import jax.numpy as jnp
from jax import lax
from jax.experimental import pallas as pl
from jax.experimental.pallas import tpu as pltpu
from jax.experimental.pallas import tpu_sc as plsc

N = 10000
E = 320000
IN_FEATS = 128
HID = 32

NC = 2
NS = 16
NW = NC * NS
CH = 128
CPT = 79
EPT = CH * CPT
E_PAD = NW * EPT
N_PAD = 10240
ROWS_PER_TILE = N_PAD // NS
STAGE_ROWS = 160
N_SUB = ROWS_PER_TILE // STAGE_ROWS

_mesh = plsc.VectorSubcoreMesh(core_axis_name="c", subcore_axis_name="s")


@functools.partial(
    pl.kernel,
    mesh=_mesh,
    compiler_params=pltpu.CompilerParams(use_tc_tiling_on_sc=False),
    out_type=jax.ShapeDtypeStruct((NC, N_PAD, HID), jnp.float32),
    scratch_types=[
        pltpu.VMEM((CPT, CH), jnp.int32),
        pltpu.VMEM((CPT, CH), jnp.int32),
        pltpu.VMEM((CH, HID), jnp.float32),
        pltpu.VMEM((CH, HID), jnp.float32),
        pltpu.VMEM((STAGE_ROWS, HID), jnp.float32),
        pltpu.VMEM_SHARED((N_PAD, HID), jnp.float32),
        pltpu.SemaphoreType.DMA,
        pltpu.SemaphoreType.DMA,
    ],
)
def _sc_message_pass(h_hbm, src_hbm, dst_hbm, zeros_hbm, out_hbm,
                     srcv, dstv, rows_a, rows_b, stage_v, agg_sh,
                     sem_a, sem_b):
    cid = lax.axis_index("c")
    sid = lax.axis_index("s")
    wid = sid * NC + cid
    r0 = sid * ROWS_PER_TILE
    pltpu.sync_copy(zeros_hbm, stage_v)
    for k in range(N_SUB):
        pltpu.sync_copy(stage_v,
                        agg_sh.at[pl.ds(r0 + k * STAGE_ROWS, STAGE_ROWS)])
    pltpu.sync_copy(src_hbm.at[wid], srcv)
    pltpu.sync_copy(dst_hbm.at[wid], dstv)
    plsc.subcore_barrier()

    pltpu.make_async_copy(h_hbm.at[srcv.at[0]], rows_a, sem_a).start()

    def pair(i, c):
        j = 1 + 2 * i
        pltpu.make_async_copy(h_hbm.at[srcv.at[j]], rows_b, sem_b).start()
        pltpu.make_async_copy(h_hbm.at[srcv.at[j - 1]], rows_a, sem_a).wait()
        pltpu.sync_copy(rows_a, agg_sh.at[dstv.at[j - 1]], add=True)
        pltpu.make_async_copy(h_hbm.at[srcv.at[j + 1]], rows_a, sem_a).start()
        pltpu.make_async_copy(h_hbm.at[srcv.at[j]], rows_b, sem_b).wait()
        pltpu.sync_copy(rows_b, agg_sh.at[dstv.at[j]], add=True)
        return c

    lax.fori_loop(0, (CPT - 1) // 2, pair, 0)
    pltpu.make_async_copy(h_hbm.at[srcv.at[CPT - 1]], rows_a, sem_a).wait()
    pltpu.sync_copy(rows_a, agg_sh.at[dstv.at[CPT - 1]], add=True)
    plsc.subcore_barrier()
    for k in range(N_SUB):
        sub = pl.ds(r0 + k * STAGE_ROWS, STAGE_ROWS)
        pltpu.sync_copy(agg_sh.at[sub], stage_v)
        pltpu.sync_copy(stage_v, out_hbm.at[cid, sub])


RB = 1000


def _tc1_body(x_ref, do_ref, di_ref, w_ref, h_ref, norm_ref):
    do_p = do_ref[...]
    di_p = di_ref[...]
    dout = (do_p[0] + do_p[1])[:, 0:1]
    din = (di_p[0] + di_p[1])[:, 0:1]
    nsrc = jnp.where(dout > 0.0, lax.rsqrt(jnp.maximum(dout, 1.0)), 0.0)
    ndst = jnp.where(din > 0.0, lax.rsqrt(jnp.maximum(din, 1.0)), 0.0)
    norm_ref[...] = jnp.concatenate([nsrc, ndst], axis=1)
    xs = x_ref[...] * nsrc
    h_ref[...] = jnp.dot(xs, w_ref[...], preferred_element_type=jnp.float32)


def _tc2_body(p_ref, norm_ref, b1_ref, w2_ref, h2_ref):
    p = p_ref[...]
    norm = norm_ref[...]
    h = (p[0] + p[1]) * norm[:, 1:2] + b1_ref[...]
    h = jnp.maximum(h, 0.0)
    h = h * norm[:, 0:1]
    h2_ref[...] = jnp.dot(h, w2_ref[...], preferred_element_type=jnp.float32)


def _tc3_body(p_ref, norm_ref, b2_ref, o_ref):
    p = p_ref[...]
    o_ref[...] = (p[0] + p[1]) * norm_ref[...][:, 1:2] + b2_ref[...]


_G = N // RB

_tc1 = pl.pallas_call(
    _tc1_body,
    grid=(_G,),
    in_specs=[
        pl.BlockSpec((RB, IN_FEATS), lambda i: (i, 0)),
        pl.BlockSpec((NC, RB, HID), lambda i: (0, i, 0)),
        pl.BlockSpec((NC, RB, HID), lambda i: (0, i, 0)),
        pl.BlockSpec((IN_FEATS, HID), lambda i: (0, 0)),
    ],
    out_specs=[
        pl.BlockSpec((RB, HID), lambda i: (i, 0)),
        pl.BlockSpec((RB, 2), lambda i: (i, 0)),
    ],
    out_shape=[
        jax.ShapeDtypeStruct((N_PAD, HID), jnp.float32),
        jax.ShapeDtypeStruct((N, 2), jnp.float32),
    ],
)

_tc2 = pl.pallas_call(
    _tc2_body,
    grid=(_G,),
    in_specs=[
        pl.BlockSpec((NC, RB, HID), lambda i: (0, i, 0)),
        pl.BlockSpec((RB, 2), lambda i: (i, 0)),
        pl.BlockSpec((1, HID), lambda i: (0, 0)),
        pl.BlockSpec((HID, HID), lambda i: (0, 0)),
    ],
    out_specs=pl.BlockSpec((RB, HID), lambda i: (i, 0)),
    out_shape=jax.ShapeDtypeStruct((N_PAD, HID), jnp.float32),
)

_tc3 = pl.pallas_call(
    _tc3_body,
    grid=(_G,),
    in_specs=[
        pl.BlockSpec((NC, RB, HID), lambda i: (0, i, 0)),
        pl.BlockSpec((RB, 2), lambda i: (i, 0)),
        pl.BlockSpec((1, HID), lambda i: (0, 0)),
    ],
    out_specs=pl.BlockSpec((RB, HID), lambda i: (i, 0)),
    out_shape=jax.ShapeDtypeStruct((N, HID), jnp.float32),
)


def kernel(x, edge_index, W1, b1, W2, b2):
    src = edge_index[0]
    dst = edge_index[1]
    pad_idx = (jnp.arange(E_PAD - E, dtype=jnp.int32) % (N_PAD - N)) + N
    src_p = jnp.concatenate([src, pad_idx]).reshape(NW, CPT, CH)
    dst_p = jnp.concatenate([dst, pad_idx]).reshape(NW, CPT, CH)

    zeros_f = jnp.zeros((STAGE_ROWS, HID), jnp.float32)
    ones_t = jnp.ones((N_PAD, HID), jnp.float32)

    din_f = _sc_message_pass(ones_t, src_p, dst_p, zeros_f)
    dout_f = _sc_message_pass(ones_t, dst_p, src_p, zeros_f)

    h1, norms = _tc1(x, dout_f, din_f, W1)
    agg1 = _sc_message_pass(h1, src_p, dst_p, zeros_f)
    h2 = _tc2(agg1, norms, b1.reshape(1, HID), W2)
    agg2 = _sc_message_pass(h2, src_p, dst_p, zeros_f)
    out = _tc3(agg2, norms, b2.reshape(1, HID))
    return out
